# Initial kernel scaffold; baseline (speedup 1.0000x reference)
import jax
import jax.numpy as jnp
from jax.experimental import pallas as pl


def kernel(x, edge_index, batch_ids, params):
    raise NotImplementedError("write your pallas kernel here")


# speedup vs baseline: 4.1141x; 4.1141x over previous
import functools

import jax
import jax.numpy as jnp
from jax import lax
from jax.experimental import pallas as pl
from jax.experimental.pallas import tpu as pltpu
from jax.experimental.pallas import tpu_sc as plsc

_LRELU = 0.2


def _leaky(v):
    return jnp.where(v > 0, v, _LRELU * v)


def _mm_body(x_ref, w_ref, b_ref, o_ref):
    o_ref[...] = (
        jnp.dot(x_ref[...], w_ref[...], preferred_element_type=jnp.float32)
        + b_ref[...]
    )


def _mm(x, W, b, bm=2048):
    M, K = x.shape
    No = W.shape[1]
    return pl.pallas_call(
        _mm_body,
        grid=(pl.cdiv(M, bm),),
        in_specs=[
            pl.BlockSpec((bm, K), lambda i: (i, 0)),
            pl.BlockSpec((K, No), lambda i: (0, 0)),
            pl.BlockSpec((1, No), lambda i: (0, 0)),
        ],
        out_specs=pl.BlockSpec((bm, No), lambda i: (i, 0)),
        out_shape=jax.ShapeDtypeStruct((M, No), jnp.float32),
    )(x, W, b.reshape(1, No))


def _edge_body(gl_ref, gr_ref, att_ref, w_ref, ee_ref, *, heads, out_c):
    gl = gl_ref[...]
    s = gl + gr_ref[...]
    s = jnp.where(s > 0, s, _LRELU * s)
    att = att_ref[...]
    for h in range(heads):
        sh = s[:, h * out_c:(h + 1) * out_c] * att[h:h + 1, :]
        ee = jnp.exp(jnp.sum(sh, axis=1, keepdims=True))
        w_ref[:, h * out_c:(h + 1) * out_c] = gl[:, h * out_c:(h + 1) * out_c] * ee
        ee_ref[:, h:h + 1] = ee


def _edge(gl, gr, att, heads, out_c, bm=4096):
    E_, F = gl.shape
    return pl.pallas_call(
        functools.partial(_edge_body, heads=heads, out_c=out_c),
        grid=(pl.cdiv(E_, bm),),
        in_specs=[
            pl.BlockSpec((bm, F), lambda i: (i, 0)),
            pl.BlockSpec((bm, F), lambda i: (i, 0)),
            pl.BlockSpec((heads, out_c), lambda i: (0, 0)),
        ],
        out_specs=[
            pl.BlockSpec((bm, F), lambda i: (i, 0)),
            pl.BlockSpec((bm, heads), lambda i: (i, 0)),
        ],
        out_shape=[
            jax.ShapeDtypeStruct((E_, F), jnp.float32),
            jax.ShapeDtypeStruct((E_, heads), jnp.float32),
        ],
    )(gl, gr, att)


def _sc_gather(table, idx, chunk=256):
    Nrows, D0 = table.shape
    D = ((D0 + 127) // 128) * 128
    if D != D0:
        table = jnp.pad(table, ((0, 0), (0, D - D0)))
    E_ = idx.shape[0]
    info = plsc.get_sparse_core_info()
    NW = info.num_cores * info.num_subcores
    n_chunks = pl.cdiv(E_, NW * chunk)
    per_w = n_chunks * chunk
    Ep = per_w * NW
    idx_p = jnp.concatenate(
        [idx, jnp.zeros((Ep - E_,), jnp.int32)]) if Ep != E_ else idx
    mesh = plsc.VectorSubcoreMesh(core_axis_name="c", subcore_axis_name="s")

    @functools.partial(
        pl.kernel,
        mesh=mesh,
        out_type=jax.ShapeDtypeStruct((Ep, D), jnp.float32),
        scratch_types=[
            pltpu.VMEM((per_w,), jnp.int32),
            pltpu.VMEM((chunk, D), jnp.float32),
            pltpu.SemaphoreType.DMA,
        ],
    )
    def k(table_hbm, idx_hbm, out_hbm, idx_v, rows_v, sem):
        wid = lax.axis_index("s") * info.num_cores + lax.axis_index("c")
        base = wid * per_w
        pltpu.sync_copy(idx_hbm.at[pl.ds(base, per_w)], idx_v)

        def body(i, carry):
            pltpu.async_copy(
                table_hbm.at[idx_v.at[pl.ds(i * chunk, chunk)]], rows_v, sem
            ).wait()
            pltpu.sync_copy(rows_v, out_hbm.at[pl.ds(base + i * chunk, chunk)])
            return carry

        lax.fori_loop(0, n_chunks, body, 0)

    out = k(table, idx_p)
    return out[:E_, :D0]


def _gatv2(p, x, src, dst, heads, out_c, n):
    F = heads * out_c
    xl = _mm(x, p["Wl"], p["bl"])
    xr = _mm(x, p["Wr"], p["br"])
    gl = _sc_gather(xl, src)
    gr = _sc_gather(xr, dst)
    w, ee = _edge(gl, gr, p["att"], heads, out_c)
    den = jax.ops.segment_sum(ee, dst, num_segments=n)
    ub = jax.ops.segment_sum(w, dst, num_segments=n)
    out = ub.reshape(n, heads, out_c) / (den + 1e-16).reshape(n, heads, 1)
    return out.reshape(n, F) + p["bias"]


def _bn(p, x):
    m = x.mean(0)
    v = x.var(0)
    return (x - m) / jnp.sqrt(v + 1e-5) * p["g"] + p["b"]


def _lin(p, x):
    return _mm(x, p["W"], p["b"])


def kernel(x, edge_index, batch_ids, params):
    n = x.shape[0]
    B = 100
    N_PER = n // B
    BANK = 10
    n_mol = N_PER - BANK
    loop = jnp.arange(n, dtype=edge_index.dtype)
    src = jnp.concatenate([edge_index[0], loop])
    dst = jnp.concatenate([edge_index[1], loop])

    px = _leaky(_bn(params["p_bnorm1"], _gatv2(params["p_conv1"], x, src, dst, 3, 64, n)))
    px = _leaky(_bn(params["p_bnorm2"], _gatv2(params["p_conv2"], px, src, dst, 1, 64, n)))
    px = _leaky(_bn(params["p_bnorm3"], _gatv2(params["p_conv3"], px, src, dst, 1, 32, n)))
    px_g = px.reshape(B, N_PER, 32)

    mol = px_g[:, :n_mol, :].reshape(B * n_mol, 32)
    h = _leaky(_lin(params["nmol_fcn1"], mol))
    h = _leaky(_lin(params["nmol_fcn2"], h))
    h = _leaky(_lin(params["nmol_fcn3"], h))
    nmol = jax.nn.softmax(h.reshape(B, n_mol, 1), axis=1).reshape(B * n_mol, 1)

    bank_fill = jnp.full((B, BANK, 1), -1.0, dtype=jnp.float32)
    nmol_full = jnp.concatenate(
        [nmol.reshape(B, n_mol, 1), bank_fill], axis=1).reshape(B * N_PER, 1)
    pxf = jnp.concatenate([px, nmol_full], axis=1)
    h = _leaky(_lin(params["nfull_fcn1"], pxf))
    h = _leaky(_lin(params["nfull_fcn2"], h))
    nfull = jax.nn.softmax(h.reshape(B, N_PER, 1), axis=1).reshape(B * N_PER, 1)

    nmol_sm = jax.nn.softmax(nmol, axis=0)
    px_nmol = (mol * nmol_sm).reshape(B, n_mol, 32).sum(axis=1)
    px_nfull = (px * nfull).reshape(B, N_PER, 32).sum(axis=1)
    p_mean = px_g.mean(axis=1)
    p_bond = jnp.concatenate([p_mean, px_nmol, px_nfull], axis=1)
    b = _leaky(_lin(params["b_fcn1"], p_bond))
    b = _leaky(_lin(params["b_fcn2"], b))
    b = jax.nn.softmax(b, axis=1)

    nm = nmol.reshape(B, n_mol)
    nf = nfull.reshape(B, N_PER)
    nmol_mean = nm.mean(axis=1, keepdims=True)
    nmol_std = jnp.std(nm, axis=1, ddof=1).reshape(B, 1)
    nfull_mean = nf.mean(axis=1, keepdims=True)
    nfull_std = jnp.std(nf, axis=1, ddof=1).reshape(B, 1)
    b_mean = b.mean(axis=1, keepdims=True)
    b_std = jnp.std(b, axis=1, ddof=1).reshape(B, 1)
    nn_col = jnp.full((B, 1), float(n_mol), dtype=jnp.float32)
    p_t = jnp.concatenate(
        [p_bond, b, nmol_mean, nmol_std, nfull_mean, nfull_std, b_mean, b_std, nn_col],
        axis=1)
    t = jax.nn.softmax(_leaky(_lin(params["t_fcn1"], p_t)), axis=1)

    vx = _leaky(_gatv2(params["v_conv1"], x, src, dst, 1, 64, n))
    vx = _leaky(_gatv2(params["v_conv2"], vx, src, dst, 1, 64, n))
    vx = _leaky(_gatv2(params["v_conv3"], vx, src, dst, 1, 32, n))
    v = vx.reshape(B, N_PER, 32).mean(axis=1)
    v = _leaky(_lin(params["v_fcn1"], v))
    v = _leaky(_lin(params["v_fcn2"], v))
    return (t, nmol, nfull, b, v)


# speedup vs baseline: 5.3531x; 1.3012x over previous
import functools

import jax
import jax.numpy as jnp
from jax import lax
from jax.experimental import pallas as pl
from jax.experimental.pallas import tpu as pltpu
from jax.experimental.pallas import tpu_sc as plsc

_LRELU = 0.2


def _leaky(v):
    return jnp.where(v > 0, v, _LRELU * v)


def _mm_body(x_ref, w_ref, b_ref, o_ref):
    o_ref[...] = (
        jnp.dot(x_ref[...], w_ref[...], preferred_element_type=jnp.float32)
        + b_ref[...]
    )


def _mm(x, W, b, bm=2048):
    M, K = x.shape
    No = W.shape[1]
    return pl.pallas_call(
        _mm_body,
        grid=(pl.cdiv(M, bm),),
        in_specs=[
            pl.BlockSpec((bm, K), lambda i: (i, 0)),
            pl.BlockSpec((K, No), lambda i: (0, 0)),
            pl.BlockSpec((1, No), lambda i: (0, 0)),
        ],
        out_specs=pl.BlockSpec((bm, No), lambda i: (i, 0)),
        out_shape=jax.ShapeDtypeStruct((M, No), jnp.float32),
    )(x, W, b.reshape(1, No))


def _edge_body(gl_ref, gr_ref, att_ref, o_ref, *, heads, out_c):
    gl = gl_ref[...]
    s = gl + gr_ref[...]
    s = jnp.where(s > 0, s, _LRELU * s)
    att = att_ref[...]
    F = heads * out_c
    for h in range(heads):
        sh = s[:, h * out_c:(h + 1) * out_c] * att[h:h + 1, :]
        ee = jnp.exp(jnp.sum(sh, axis=1, keepdims=True))
        o_ref[:, h * out_c:(h + 1) * out_c] = gl[:, h * out_c:(h + 1) * out_c] * ee
        o_ref[:, F + h:F + h + 1] = ee


def _edge(gl, gr, att, heads, out_c, bm=4096):
    E_, F = gl.shape
    return pl.pallas_call(
        functools.partial(_edge_body, heads=heads, out_c=out_c),
        grid=(pl.cdiv(E_, bm),),
        in_specs=[
            pl.BlockSpec((bm, F), lambda i: (i, 0)),
            pl.BlockSpec((bm, F), lambda i: (i, 0)),
            pl.BlockSpec((heads, out_c), lambda i: (0, 0)),
        ],
        out_specs=pl.BlockSpec((bm, F + heads), lambda i: (i, 0)),
        out_shape=jax.ShapeDtypeStruct((E_, F + heads), jnp.float32),
    )(gl, gr, att)


def _sc_gather2(tl, tr, src, dst):
    Nrows, D0 = tl.shape
    D = ((D0 + 127) // 128) * 128
    if D != D0:
        tl = jnp.pad(tl, ((0, 0), (0, D - D0)))
        tr = jnp.pad(tr, ((0, 0), (0, D - D0)))
    chunk = 128 if D >= 256 else 256
    E_ = src.shape[0]
    info = plsc.get_sparse_core_info()
    NW = info.num_cores * info.num_subcores
    n_chunks = pl.cdiv(E_, NW * chunk)
    per_w = n_chunks * chunk
    Ep = per_w * NW
    if Ep != E_:
        z = jnp.zeros((Ep - E_,), jnp.int32)
        src = jnp.concatenate([src, z])
        dst = jnp.concatenate([dst, z])
    mesh = plsc.VectorSubcoreMesh(core_axis_name="c", subcore_axis_name="s")

    @functools.partial(
        pl.kernel,
        mesh=mesh,
        out_type=[
            jax.ShapeDtypeStruct((Ep, D), jnp.float32),
            jax.ShapeDtypeStruct((Ep, D), jnp.float32),
        ],
        scratch_types=[
            pltpu.VMEM((per_w,), jnp.int32),
            pltpu.VMEM((per_w,), jnp.int32),
            pltpu.VMEM((chunk, D), jnp.float32),
            pltpu.VMEM((chunk, D), jnp.float32),
            pltpu.SemaphoreType.DMA,
            pltpu.SemaphoreType.DMA,
        ],
    )
    def k(tl_hbm, tr_hbm, src_hbm, dst_hbm, ol_hbm, or_hbm,
          si_v, di_v, ra, rb, sa, sb):
        wid = lax.axis_index("s") * info.num_cores + lax.axis_index("c")
        base = wid * per_w
        pltpu.sync_copy(src_hbm.at[pl.ds(base, per_w)], si_v)
        pltpu.sync_copy(dst_hbm.at[pl.ds(base, per_w)], di_v)

        def body(i, carry):
            off = i * chunk
            ca = pltpu.async_copy(tl_hbm.at[si_v.at[pl.ds(off, chunk)]], ra, sa)
            cb = pltpu.async_copy(tr_hbm.at[di_v.at[pl.ds(off, chunk)]], rb, sb)
            ca.wait()
            pltpu.sync_copy(ra, ol_hbm.at[pl.ds(base + off, chunk)])
            cb.wait()
            pltpu.sync_copy(rb, or_hbm.at[pl.ds(base + off, chunk)])
            return carry

        lax.fori_loop(0, n_chunks, body, 0)

    gl, gr = k(tl, tr, src, dst)
    return gl[:E_, :D0], gr[:E_, :D0]


def _gatv2(p, x, src, dst, heads, out_c, n):
    F = heads * out_c
    xl = _mm(x, p["Wl"], p["bl"])
    xr = _mm(x, p["Wr"], p["br"])
    gl, gr = _sc_gather2(xl, xr, src, dst)
    wee = _edge(gl, gr, p["att"], heads, out_c)
    seg = jax.ops.segment_sum(wee, dst, num_segments=n)
    ub, den = seg[:, :F], seg[:, F:]
    out = ub.reshape(n, heads, out_c) / (den + 1e-16).reshape(n, heads, 1)
    return out.reshape(n, F) + p["bias"]


def _bn(p, x):
    m = x.mean(0)
    v = x.var(0)
    return (x - m) / jnp.sqrt(v + 1e-5) * p["g"] + p["b"]


def _lin(p, x):
    return _mm(x, p["W"], p["b"])


def kernel(x, edge_index, batch_ids, params):
    n = x.shape[0]
    B = 100
    N_PER = n // B
    BANK = 10
    n_mol = N_PER - BANK
    loop = jnp.arange(n, dtype=edge_index.dtype)
    src = jnp.concatenate([edge_index[0], loop])
    dst = jnp.concatenate([edge_index[1], loop])

    px = _leaky(_bn(params["p_bnorm1"], _gatv2(params["p_conv1"], x, src, dst, 3, 64, n)))
    px = _leaky(_bn(params["p_bnorm2"], _gatv2(params["p_conv2"], px, src, dst, 1, 64, n)))
    px = _leaky(_bn(params["p_bnorm3"], _gatv2(params["p_conv3"], px, src, dst, 1, 32, n)))
    px_g = px.reshape(B, N_PER, 32)

    mol = px_g[:, :n_mol, :].reshape(B * n_mol, 32)
    h = _leaky(_lin(params["nmol_fcn1"], mol))
    h = _leaky(_lin(params["nmol_fcn2"], h))
    h = _leaky(_lin(params["nmol_fcn3"], h))
    nmol = jax.nn.softmax(h.reshape(B, n_mol, 1), axis=1).reshape(B * n_mol, 1)

    bank_fill = jnp.full((B, BANK, 1), -1.0, dtype=jnp.float32)
    nmol_full = jnp.concatenate(
        [nmol.reshape(B, n_mol, 1), bank_fill], axis=1).reshape(B * N_PER, 1)
    pxf = jnp.concatenate([px, nmol_full], axis=1)
    h = _leaky(_lin(params["nfull_fcn1"], pxf))
    h = _leaky(_lin(params["nfull_fcn2"], h))
    nfull = jax.nn.softmax(h.reshape(B, N_PER, 1), axis=1).reshape(B * N_PER, 1)

    nmol_sm = jax.nn.softmax(nmol, axis=0)
    px_nmol = (mol * nmol_sm).reshape(B, n_mol, 32).sum(axis=1)
    px_nfull = (px * nfull).reshape(B, N_PER, 32).sum(axis=1)
    p_mean = px_g.mean(axis=1)
    p_bond = jnp.concatenate([p_mean, px_nmol, px_nfull], axis=1)
    b = _leaky(_lin(params["b_fcn1"], p_bond))
    b = _leaky(_lin(params["b_fcn2"], b))
    b = jax.nn.softmax(b, axis=1)

    nm = nmol.reshape(B, n_mol)
    nf = nfull.reshape(B, N_PER)
    nmol_mean = nm.mean(axis=1, keepdims=True)
    nmol_std = jnp.std(nm, axis=1, ddof=1).reshape(B, 1)
    nfull_mean = nf.mean(axis=1, keepdims=True)
    nfull_std = jnp.std(nf, axis=1, ddof=1).reshape(B, 1)
    b_mean = b.mean(axis=1, keepdims=True)
    b_std = jnp.std(b, axis=1, ddof=1).reshape(B, 1)
    nn_col = jnp.full((B, 1), float(n_mol), dtype=jnp.float32)
    p_t = jnp.concatenate(
        [p_bond, b, nmol_mean, nmol_std, nfull_mean, nfull_std, b_mean, b_std, nn_col],
        axis=1)
    t = jax.nn.softmax(_leaky(_lin(params["t_fcn1"], p_t)), axis=1)

    vx = _leaky(_gatv2(params["v_conv1"], x, src, dst, 1, 64, n))
    vx = _leaky(_gatv2(params["v_conv2"], vx, src, dst, 1, 64, n))
    vx = _leaky(_gatv2(params["v_conv3"], vx, src, dst, 1, 32, n))
    v = vx.reshape(B, N_PER, 32).mean(axis=1)
    v = _leaky(_lin(params["v_fcn1"], v))
    v = _leaky(_lin(params["v_fcn2"], v))
    return (t, nmol, nfull, b, v)


# speedup vs baseline: 7.9613x; 1.4872x over previous
import functools

import jax
import jax.numpy as jnp
from jax import lax
from jax.experimental import pallas as pl
from jax.experimental.pallas import tpu as pltpu
from jax.experimental.pallas import tpu_sc as plsc

_LRELU = 0.2


def _leaky(v):
    return jnp.where(v > 0, v, _LRELU * v)


def _mm_body(x_ref, w_ref, b_ref, o_ref):
    o_ref[...] = (
        jnp.dot(x_ref[...], w_ref[...], preferred_element_type=jnp.float32)
        + b_ref[...]
    )


def _mm(x, W, b, bm=2048):
    M, K = x.shape
    No = W.shape[1]
    return pl.pallas_call(
        _mm_body,
        grid=(pl.cdiv(M, bm),),
        in_specs=[
            pl.BlockSpec((bm, K), lambda i: (i, 0)),
            pl.BlockSpec((K, No), lambda i: (0, 0)),
            pl.BlockSpec((1, No), lambda i: (0, 0)),
        ],
        out_specs=pl.BlockSpec((bm, No), lambda i: (i, 0)),
        out_shape=jax.ShapeDtypeStruct((M, No), jnp.float32),
    )(x, W, b.reshape(1, No))


def _edge_body(gl_ref, gr_ref, att_ref, o_ref, *, heads, out_c, e_real, bm):
    gl = gl_ref[...]
    s = gl + gr_ref[...]
    s = jnp.where(s > 0, s, _LRELU * s)
    att = att_ref[...]
    F = heads * out_c
    rid = pl.program_id(0) * bm + lax.broadcasted_iota(jnp.int32, (bm, 1), 0)
    valid = rid < e_real
    for h in range(heads):
        sh = s[:, h * out_c:(h + 1) * out_c] * att[h:h + 1, :]
        ee = jnp.exp(jnp.sum(sh, axis=1, keepdims=True))
        ee = jnp.where(valid, ee, 0.0)
        o_ref[:, h * out_c:(h + 1) * out_c] = gl[:, h * out_c:(h + 1) * out_c] * ee
        o_ref[:, F + h:F + h + 1] = ee


def _edge(gl, gr, att, heads, out_c, e_real, bm=4096):
    E_, D = gl.shape
    F = heads * out_c
    return pl.pallas_call(
        functools.partial(_edge_body, heads=heads, out_c=out_c,
                          e_real=e_real, bm=bm),
        grid=(pl.cdiv(E_, bm),),
        in_specs=[
            pl.BlockSpec((bm, D), lambda i: (i, 0)),
            pl.BlockSpec((bm, D), lambda i: (i, 0)),
            pl.BlockSpec((heads, out_c), lambda i: (0, 0)),
        ],
        out_specs=pl.BlockSpec((bm, F + heads), lambda i: (i, 0)),
        out_shape=jax.ShapeDtypeStruct((E_, F + heads), jnp.float32),
    )(gl, gr, att)


def _sc_gather2(tl, tr, src, dst):
    Nrows, D0 = tl.shape
    D = ((D0 + 127) // 128) * 128
    if D != D0:
        tl = jnp.pad(tl, ((0, 0), (0, D - D0)))
        tr = jnp.pad(tr, ((0, 0), (0, D - D0)))
    chunk = 128 if D >= 256 else 256
    E_ = src.shape[0]
    info = plsc.get_sparse_core_info()
    NW = info.num_cores * info.num_subcores
    n_chunks = pl.cdiv(E_, NW * chunk)
    per_w = n_chunks * chunk
    Ep = per_w * NW
    if Ep != E_:
        z = jnp.zeros((Ep - E_,), jnp.int32)
        src = jnp.concatenate([src, z])
        dst = jnp.concatenate([dst, z])
    mesh = plsc.VectorSubcoreMesh(core_axis_name="c", subcore_axis_name="s")

    @functools.partial(
        pl.kernel,
        mesh=mesh,
        out_type=[
            jax.ShapeDtypeStruct((Ep, D), jnp.float32),
            jax.ShapeDtypeStruct((Ep, D), jnp.float32),
        ],
        scratch_types=[
            pltpu.VMEM((per_w,), jnp.int32),
            pltpu.VMEM((per_w,), jnp.int32),
            pltpu.VMEM((chunk, D), jnp.float32),
            pltpu.VMEM((chunk, D), jnp.float32),
            pltpu.SemaphoreType.DMA,
            pltpu.SemaphoreType.DMA,
        ],
    )
    def k(tl_hbm, tr_hbm, src_hbm, dst_hbm, ol_hbm, or_hbm,
          si_v, di_v, ra, rb, sa, sb):
        wid = lax.axis_index("s") * info.num_cores + lax.axis_index("c")
        base = wid * per_w
        pltpu.sync_copy(src_hbm.at[pl.ds(base, per_w)], si_v)
        pltpu.sync_copy(dst_hbm.at[pl.ds(base, per_w)], di_v)

        def body(i, carry):
            off = i * chunk
            ca = pltpu.async_copy(tl_hbm.at[si_v.at[pl.ds(off, chunk)]], ra, sa)
            cb = pltpu.async_copy(tr_hbm.at[di_v.at[pl.ds(off, chunk)]], rb, sb)
            ca.wait()
            pltpu.sync_copy(ra, ol_hbm.at[pl.ds(base + off, chunk)])
            cb.wait()
            pltpu.sync_copy(rb, or_hbm.at[pl.ds(base + off, chunk)])
            return carry

        lax.fori_loop(0, n_chunks, body, 0)

    gl, gr = k(tl, tr, src, dst)
    return gl, gr, Ep


def _gatv2(p, x, src, dst, heads, out_c, n):
    F = heads * out_c
    xl = _mm(x, p["Wl"], p["bl"])
    xr = _mm(x, p["Wr"], p["br"])
    gl, gr, Ep = _sc_gather2(xl, xr, src, dst)
    E_ = dst.shape[0]
    wee = _edge(gl, gr, p["att"], heads, out_c, E_)
    if Ep != E_:
        dst = jnp.concatenate(
            [dst, jnp.full((Ep - E_,), n, dst.dtype)])
    seg = jax.ops.segment_sum(wee, dst, num_segments=n)
    ub, den = seg[:, :F], seg[:, F:]
    out = ub.reshape(n, heads, out_c) / (den + 1e-16).reshape(n, heads, 1)
    return out.reshape(n, F) + p["bias"]


def _bn(p, x):
    m = x.mean(0)
    v = x.var(0)
    return (x - m) / jnp.sqrt(v + 1e-5) * p["g"] + p["b"]


def _lin(p, x):
    return _mm(x, p["W"], p["b"])


def kernel(x, edge_index, batch_ids, params):
    n = x.shape[0]
    B = 100
    N_PER = n // B
    BANK = 10
    n_mol = N_PER - BANK
    loop = jnp.arange(n, dtype=edge_index.dtype)
    src = jnp.concatenate([edge_index[0], loop])
    dst = jnp.concatenate([edge_index[1], loop])

    px = _leaky(_bn(params["p_bnorm1"], _gatv2(params["p_conv1"], x, src, dst, 3, 64, n)))
    px = _leaky(_bn(params["p_bnorm2"], _gatv2(params["p_conv2"], px, src, dst, 1, 64, n)))
    px = _leaky(_bn(params["p_bnorm3"], _gatv2(params["p_conv3"], px, src, dst, 1, 32, n)))
    px_g = px.reshape(B, N_PER, 32)

    mol = px_g[:, :n_mol, :].reshape(B * n_mol, 32)
    h = _leaky(_lin(params["nmol_fcn1"], mol))
    h = _leaky(_lin(params["nmol_fcn2"], h))
    h = _leaky(_lin(params["nmol_fcn3"], h))
    nmol = jax.nn.softmax(h.reshape(B, n_mol, 1), axis=1).reshape(B * n_mol, 1)

    bank_fill = jnp.full((B, BANK, 1), -1.0, dtype=jnp.float32)
    nmol_full = jnp.concatenate(
        [nmol.reshape(B, n_mol, 1), bank_fill], axis=1).reshape(B * N_PER, 1)
    pxf = jnp.concatenate([px, nmol_full], axis=1)
    h = _leaky(_lin(params["nfull_fcn1"], pxf))
    h = _leaky(_lin(params["nfull_fcn2"], h))
    nfull = jax.nn.softmax(h.reshape(B, N_PER, 1), axis=1).reshape(B * N_PER, 1)

    nmol_sm = jax.nn.softmax(nmol, axis=0)
    px_nmol = (mol * nmol_sm).reshape(B, n_mol, 32).sum(axis=1)
    px_nfull = (px * nfull).reshape(B, N_PER, 32).sum(axis=1)
    p_mean = px_g.mean(axis=1)
    p_bond = jnp.concatenate([p_mean, px_nmol, px_nfull], axis=1)
    b = _leaky(_lin(params["b_fcn1"], p_bond))
    b = _leaky(_lin(params["b_fcn2"], b))
    b = jax.nn.softmax(b, axis=1)

    nm = nmol.reshape(B, n_mol)
    nf = nfull.reshape(B, N_PER)
    nmol_mean = nm.mean(axis=1, keepdims=True)
    nmol_std = jnp.std(nm, axis=1, ddof=1).reshape(B, 1)
    nfull_mean = nf.mean(axis=1, keepdims=True)
    nfull_std = jnp.std(nf, axis=1, ddof=1).reshape(B, 1)
    b_mean = b.mean(axis=1, keepdims=True)
    b_std = jnp.std(b, axis=1, ddof=1).reshape(B, 1)
    nn_col = jnp.full((B, 1), float(n_mol), dtype=jnp.float32)
    p_t = jnp.concatenate(
        [p_bond, b, nmol_mean, nmol_std, nfull_mean, nfull_std, b_mean, b_std, nn_col],
        axis=1)
    t = jax.nn.softmax(_leaky(_lin(params["t_fcn1"], p_t)), axis=1)

    vx = _leaky(_gatv2(params["v_conv1"], x, src, dst, 1, 64, n))
    vx = _leaky(_gatv2(params["v_conv2"], vx, src, dst, 1, 64, n))
    vx = _leaky(_gatv2(params["v_conv3"], vx, src, dst, 1, 32, n))
    v = vx.reshape(B, N_PER, 32).mean(axis=1)
    v = _leaky(_lin(params["v_fcn1"], v))
    v = _leaky(_lin(params["v_fcn2"], v))
    return (t, nmol, nfull, b, v)
